# baseline (speedup 1.0000x reference)
import jax
import jax.numpy as jnp
from jax.experimental import pallas as pl
from jax.experimental.pallas import tpu as pltpu

_ROTS = ((13, 15, 26, 6), (17, 29, 16, 24))


def _rotl(x, r):
    return (x << jnp.uint32(r)) | (x >> jnp.uint32(32 - r))


def _threefry_bits(lo):
    ks = (jnp.uint32(0), jnp.uint32(42), jnp.uint32(0x1BD11BDA ^ 42))
    x0 = jnp.zeros_like(lo) + ks[0]
    x1 = lo + ks[1]
    for i in range(5):
        for r in _ROTS[i % 2]:
            x0 = x0 + x1
            x1 = _rotl(x1, r)
            x1 = x0 ^ x1
        x0 = x0 + ks[(i + 1) % 3]
        x1 = x1 + ks[(i + 2) % 3] + jnp.uint32(i + 1)
    return x0 ^ x1


def _exp_noise(flat_idx_u32):
    bits = _threefry_bits(flat_idx_u32)
    fb = (bits >> jnp.uint32(9)) | jnp.uint32(0x3F800000)
    u = jax.lax.bitcast_convert_type(fb, jnp.float32) - jnp.float32(1.0)
    return jnp.maximum(-jnp.log1p(-u), jnp.float32(1e-10))


def _make_body(R, V, CW):
    n_chunks, tail = divmod(V, CW)

    def body(t_ref, x_ref, o_ref):
        t = jnp.clip(t_ref[...], 1e-8, None)
        m = jnp.max(x_ref[...] / t, axis=-1, keepdims=True)
        base = pl.program_id(0) * (R * V)
        rowoff = jax.lax.broadcasted_iota(jnp.int32, (R, CW), 0) * V
        coloff = jax.lax.broadcasted_iota(jnp.int32, (R, CW), 1)

        def score_chunk(col0, width):
            xc = x_ref[:, pl.ds(col0, width)] / t
            ec = jnp.exp(xc - m)
            flat = (base + col0 + rowoff[:, :width]
                    + coloff[:, :width]).astype(jnp.uint32)
            s = ec / _exp_noise(flat)
            cmax = jnp.max(s, axis=-1, keepdims=True)
            carg = jnp.argmax(s, axis=-1)[:, None].astype(jnp.int32) + col0
            return cmax, carg

        def step(k, carry):
            best, bidx = carry
            cmax, carg = score_chunk(k * CW, CW)
            upd = cmax > best
            return (jnp.where(upd, cmax, best),
                    jnp.where(upd, carg, bidx))

        best = jnp.full((R, 1), -1.0, jnp.float32)
        bidx = jnp.zeros((R, 1), jnp.int32)
        best, bidx = jax.lax.fori_loop(0, n_chunks, step, (best, bidx))
        if tail:
            cmax, carg = score_chunk(n_chunks * CW, tail)
            upd = cmax > best
            bidx = jnp.where(upd, carg, bidx)
        o_ref[...] = bidx

    return body


def kernel(logits, temperatures):
    B, V = logits.shape
    R = 8
    CW = 1024
    out = pl.pallas_call(
        _make_body(R, V, CW),
        grid=(B // R,),
        in_specs=[
            pl.BlockSpec((R, 1), lambda i: (i, 0)),
            pl.BlockSpec((R, V), lambda i: (i, 0)),
        ],
        out_specs=pl.BlockSpec((R, 1), lambda i: (i, 0)),
        out_shape=jax.ShapeDtypeStruct((B, 1), jnp.int32),
        compiler_params=pltpu.CompilerParams(
            dimension_semantics=("arbitrary",)),
    )(temperatures[:, None], logits)
    return out[:, 0]


# speedup vs baseline: 1.7949x; 1.7949x over previous
import jax
import jax.numpy as jnp
from jax.experimental import pallas as pl
from jax.experimental.pallas import tpu as pltpu

_ROTS = ((13, 15, 26, 6), (17, 29, 16, 24))


def _rotl(x, r):
    return (x << jnp.uint32(r)) | (x >> jnp.uint32(32 - r))


def _threefry_bits(lo):
    ks = (jnp.uint32(0), jnp.uint32(42), jnp.uint32(0x1BD11BDA ^ 42))
    x0 = jnp.zeros_like(lo) + ks[0]
    x1 = lo + ks[1]
    for i in range(5):
        for r in _ROTS[i % 2]:
            x0 = x0 + x1
            x1 = _rotl(x1, r)
            x1 = x0 ^ x1
        x0 = x0 + ks[(i + 1) % 3]
        x1 = x1 + ks[(i + 2) % 3] + jnp.uint32(i + 1)
    return x0 ^ x1


def _exp_noise(flat_idx_u32):
    bits = _threefry_bits(flat_idx_u32)
    fb = (bits >> jnp.uint32(9)) | jnp.uint32(0x3F800000)
    u = jax.lax.bitcast_convert_type(fb, jnp.float32) - jnp.float32(1.0)
    return jnp.maximum(-jnp.log1p(-u), jnp.float32(1e-10))


def _make_body(R, V, CW):
    n_chunks, tail = divmod(V, CW)

    def body(t_ref, x_ref, o_ref):
        t = jnp.clip(t_ref[...], 1e-8, None)
        m = jnp.max(x_ref[...] / t, axis=-1, keepdims=True)
        base = pl.program_id(0) * (R * V)
        rowoff = jax.lax.broadcasted_iota(jnp.int32, (R, CW), 0) * V
        coloff = jax.lax.broadcasted_iota(jnp.int32, (R, CW), 1)

        def score_chunk(col0, width):
            xc = x_ref[:, col0:col0 + width] / t
            ec = jnp.exp(xc - m)
            flat = (base + col0 + rowoff[:, :width]
                    + coloff[:, :width]).astype(jnp.uint32)
            s = ec / _exp_noise(flat)
            cmax = jnp.max(s, axis=-1, keepdims=True)
            carg = jnp.argmax(s, axis=-1)[:, None].astype(jnp.int32) + col0
            return cmax, carg

        best = jnp.full((R, 1), -1.0, jnp.float32)
        bidx = jnp.zeros((R, 1), jnp.int32)
        for k in range(n_chunks + (1 if tail else 0)):
            col0 = k * CW
            cmax, carg = score_chunk(col0, min(CW, V - col0))
            upd = cmax > best
            best = jnp.where(upd, cmax, best)
            bidx = jnp.where(upd, carg, bidx)
        o_ref[...] = bidx

    return body


def kernel(logits, temperatures):
    B, V = logits.shape
    R = 8
    CW = 1024
    out = pl.pallas_call(
        _make_body(R, V, CW),
        grid=(B // R,),
        in_specs=[
            pl.BlockSpec((R, 1), lambda i: (i, 0)),
            pl.BlockSpec((R, V), lambda i: (i, 0)),
        ],
        out_specs=pl.BlockSpec((R, 1), lambda i: (i, 0)),
        out_shape=jax.ShapeDtypeStruct((B, 1), jnp.int32),
        compiler_params=pltpu.CompilerParams(
            dimension_semantics=("arbitrary",)),
    )(temperatures[:, None], logits)
    return out[:, 0]
